# Initial kernel scaffold; baseline (speedup 1.0000x reference)
import jax
import jax.numpy as jnp
from jax.experimental import pallas as pl


def kernel(x, edge_index, Wq1, bq1, Wk1, bk1, Wv1, bv1, Ws1, bs1, Wq2, bq2, Wk2, bk2, Wv2, bv2, Ws2, bs2, a):
    raise NotImplementedError("write your pallas kernel here")


# speedup vs baseline: 30.2136x; 30.2136x over previous
import functools

import jax
import jax.numpy as jnp
from jax import lax
from jax.experimental import pallas as pl
from jax.experimental.pallas import tpu as pltpu
from jax.experimental.pallas import tpu_sc as plsc

N = 10000
E = 320000
D = 128
H = 8
DH = 16

NC = 2
NS = 16
NW = NC * NS
EPW = E // NW
BS = 80
NCHUNK = EPW // BS
ZB = 80
NZCH = N // ZB

_f32 = jnp.float32
BN = 1000
BE = 2000


def _qkvs_body(x_ref, wq_ref, wk_ref, wv_ref, ws_ref,
               bq_ref, bk_ref, bv_ref, bs_ref,
               q_ref, kv_ref, s_ref):
    x = x_ref[...]
    q_ref[...] = jnp.dot(x, wq_ref[...], preferred_element_type=_f32) + bq_ref[...]
    k = jnp.dot(x, wk_ref[...], preferred_element_type=_f32) + bk_ref[...]
    v = jnp.dot(x, wv_ref[...], preferred_element_type=_f32) + bv_ref[...]
    kv_ref[...] = jnp.concatenate([k, v], axis=1)
    s_ref[...] = jnp.dot(x, ws_ref[...], preferred_element_type=_f32) + bs_ref[...]


_W_SPEC = pl.BlockSpec((D, D), lambda i: (0, 0))
_B_SPEC = pl.BlockSpec((1, D), lambda i: (0, 0))

_tc_qkvs = pl.pallas_call(
    _qkvs_body,
    grid=(N // BN,),
    in_specs=[pl.BlockSpec((BN, D), lambda i: (i, 0)),
              _W_SPEC, _W_SPEC, _W_SPEC, _W_SPEC,
              _B_SPEC, _B_SPEC, _B_SPEC, _B_SPEC],
    out_specs=[pl.BlockSpec((BN, D), lambda i: (i, 0)),
               pl.BlockSpec((BN, 2 * D), lambda i: (i, 0)),
               pl.BlockSpec((BN, D), lambda i: (i, 0))],
    out_shape=[jax.ShapeDtypeStruct((N, D), _f32),
               jax.ShapeDtypeStruct((N, 2 * D), _f32),
               jax.ShapeDtypeStruct((N, D), _f32)],
)


def _gather_body(q_hbm, kv_hbm, src_hbm, dst_hbm, qe_out, kve_out,
                 dsti, srci, qbuf, kvbuf):
    c = lax.axis_index("c")
    s = lax.axis_index("s")
    base = (c * NS + s) * EPW

    def body(j, carry):
        off = base + j * BS
        pltpu.sync_copy(dst_hbm.at[pl.ds(off, BS)], dsti)
        pltpu.sync_copy(src_hbm.at[pl.ds(off, BS)], srci)
        pltpu.sync_copy(q_hbm.at[dsti], qbuf)
        pltpu.sync_copy(kv_hbm.at[srci], kvbuf)
        pltpu.sync_copy(qbuf, qe_out.at[pl.ds(off, BS)])
        pltpu.sync_copy(kvbuf, kve_out.at[pl.ds(off, BS)])
        return carry

    lax.fori_loop(0, NCHUNK, body, 0)


_sc_gather = functools.partial(
    pl.kernel,
    mesh=plsc.VectorSubcoreMesh(core_axis_name="c", subcore_axis_name="s"),
    out_type=[jax.ShapeDtypeStruct((E, D), _f32),
              jax.ShapeDtypeStruct((E, 2 * D), _f32)],
    scratch_types=[pltpu.VMEM((BS,), jnp.int32),
                   pltpu.VMEM((BS,), jnp.int32),
                   pltpu.VMEM((BS, D), _f32),
                   pltpu.VMEM((BS, 2 * D), _f32)],
)(_gather_body)


def _msg_body(qe_ref, kve_ref, msg_ref, wpad_ref):
    q = qe_ref[...]
    k = kve_ref[:, :D]
    v = kve_ref[:, D:]
    p = q * k
    r = lax.broadcasted_iota(jnp.int32, (D, H), 0) // DH
    cc = lax.broadcasted_iota(jnp.int32, (D, H), 1)
    m = jnp.where(r == cc, 0.25, 0.0).astype(_f32)
    alpha = jnp.dot(p, m, preferred_element_type=_f32)
    w = jnp.exp(alpha)
    rb = lax.broadcasted_iota(jnp.int32, (H, D), 0)
    cb = lax.broadcasted_iota(jnp.int32, (H, D), 1) // DH
    bmat = jnp.where(rb == cb, 1.0, 0.0).astype(_f32)
    wb = jnp.dot(w, bmat, preferred_element_type=_f32)
    msg_ref[...] = v * wb
    rp = lax.broadcasted_iota(jnp.int32, (H, DH), 0)
    cp = lax.broadcasted_iota(jnp.int32, (H, DH), 1)
    pmat = jnp.where(rp == cp, 1.0, 0.0).astype(_f32)
    wpad_ref[...] = jnp.dot(w, pmat, preferred_element_type=_f32)


_tc_msg = pl.pallas_call(
    _msg_body,
    grid=(E // BE,),
    in_specs=[pl.BlockSpec((BE, D), lambda i: (i, 0)),
              pl.BlockSpec((BE, 2 * D), lambda i: (i, 0))],
    out_specs=[pl.BlockSpec((BE, D), lambda i: (i, 0)),
               pl.BlockSpec((BE, DH), lambda i: (i, 0))],
    out_shape=[jax.ShapeDtypeStruct((E, D), _f32),
               jax.ShapeDtypeStruct((E, DH), _f32)],
)


def _make_scatter(width):
    def _body(rows_hbm, dst_hbm, acc_out, dsti, rbuf, zbuf, accs):
        c = lax.axis_index("c")
        s = lax.axis_index("s")

        def zrow(i, carry):
            for j in range(width // 16):
                zbuf[i, pl.ds(j * 16, 16)] = jnp.zeros((16,), _f32)
            return carry

        lax.fori_loop(0, ZB, zrow, 0)

        def zchunk(i, carry):
            ch = s + NS * i

            @pl.when(ch < NZCH)
            def _():
                pltpu.sync_copy(zbuf, accs.at[pl.ds(ch * ZB, ZB)])

            return carry

        lax.fori_loop(0, (NZCH + NS - 1) // NS, zchunk, 0)
        plsc.subcore_barrier()

        base = (c * NS + s) * EPW

        def body(j, carry):
            off = base + j * BS
            pltpu.sync_copy(dst_hbm.at[pl.ds(off, BS)], dsti)
            pltpu.sync_copy(rows_hbm.at[pl.ds(off, BS)], rbuf)
            pltpu.sync_copy(rbuf, accs.at[dsti], add=True)
            return carry

        lax.fori_loop(0, NCHUNK, body, 0)
        plsc.subcore_barrier()

        def ochunk(i, carry):
            ch = s + NS * i

            @pl.when(ch < NZCH)
            def _():
                pltpu.sync_copy(accs.at[pl.ds(ch * ZB, ZB)],
                                acc_out.at[c].at[pl.ds(ch * ZB, ZB)])

            return carry

        lax.fori_loop(0, (NZCH + NS - 1) // NS, ochunk, 0)

    return functools.partial(
        pl.kernel,
        mesh=plsc.VectorSubcoreMesh(core_axis_name="c", subcore_axis_name="s"),
        out_type=jax.ShapeDtypeStruct((NC, N, width), _f32),
        scratch_types=[pltpu.VMEM((BS,), jnp.int32),
                       pltpu.VMEM((BS, width), _f32),
                       pltpu.VMEM((ZB, width), _f32),
                       pltpu.VMEM_SHARED((N, width), _f32)],
    )(_body)


_sc_scatter_msg = _make_scatter(D)
_sc_scatter_den = _make_scatter(DH)


def _final_body(acc_ref, den_ref, s_ref, x_ref, a_ref, o_ref, *, residual):
    acc = acc_ref[0] + acc_ref[1]
    den = den_ref[0] + den_ref[1]
    rp = lax.broadcasted_iota(jnp.int32, (DH, D), 0)
    cp = lax.broadcasted_iota(jnp.int32, (DH, D), 1) // DH
    pb = jnp.where(rp == cp, 1.0, 0.0).astype(_f32)
    denb = jnp.dot(den, pb, preferred_element_type=_f32)
    out = acc / (denb + 1e-16) + s_ref[...]
    if residual:
        out = out + x_ref[...]
    a = a_ref[0, 0]
    o_ref[...] = jnp.where(out >= 0, out, a * out)


def _make_final(residual):
    return pl.pallas_call(
        functools.partial(_final_body, residual=residual),
        grid=(N // BN,),
        in_specs=[pl.BlockSpec((NC, BN, D), lambda i: (0, i, 0)),
                  pl.BlockSpec((NC, BN, DH), lambda i: (0, i, 0)),
                  pl.BlockSpec((BN, D), lambda i: (i, 0)),
                  pl.BlockSpec((BN, D), lambda i: (i, 0)),
                  pl.BlockSpec((1, 1), lambda i: (0, 0))],
        out_specs=pl.BlockSpec((BN, D), lambda i: (i, 0)),
        out_shape=jax.ShapeDtypeStruct((N, D), _f32),
    )


_tc_final_plain = _make_final(False)
_tc_final_resid = _make_final(True)


def _layer(xin, src, dst, wq, bq, wk, bk, wv, bv, ws, bs, a2, xres):
    q, kv, sk = _tc_qkvs(xin, wq, wk, wv, ws,
                         bq.reshape(1, D), bk.reshape(1, D),
                         bv.reshape(1, D), bs.reshape(1, D))
    qe, kve = _sc_gather(q, kv, src, dst)
    msg, wpad = _tc_msg(qe, kve)
    accs = _sc_scatter_msg(msg, dst)
    dens = _sc_scatter_den(wpad, dst)
    if xres is None:
        return _tc_final_plain(accs, dens, sk, xin, a2)
    return _tc_final_resid(accs, dens, sk, xres, a2)


def kernel(x, edge_index, Wq1, bq1, Wk1, bk1, Wv1, bv1, Ws1, bs1,
           Wq2, bq2, Wk2, bk2, Wv2, bv2, Ws2, bs2, a):
    src = edge_index[0].astype(jnp.int32)
    dst = edge_index[1].astype(jnp.int32)
    a2 = jnp.asarray(a, _f32).reshape(1, 1)
    x1 = _layer(x, src, dst, Wq1, bq1, Wk1, bk1, Wv1, bv1, Ws1, bs1, a2, None)
    return _layer(x1, src, dst, Wq2, bq2, Wk2, bk2, Wv2, bv2, Ws2, bs2, a2, x)
